# Initial kernel scaffold; baseline (speedup 1.0000x reference)
import jax
import jax.numpy as jnp
from jax.experimental import pallas as pl


def kernel(h, edge_index, W1, al1, ar1, b1, W2, al2, ar2, b2):
    raise NotImplementedError("write your pallas kernel here")


# speedup vs baseline: 47.9798x; 47.9798x over previous
import functools

import jax
import jax.numpy as jnp
from jax import lax
from jax.experimental import pallas as pl
from jax.experimental.pallas import tpu as pltpu
from jax.experimental.pallas import tpu_sc as plsc

N = 10000
E = 320000
D_IN = 128
H1, F1 = 8, 8
HF1 = H1 * F1
F2 = 40
NEG_SLOPE = 0.2

NWORK = 32
EPT = E // NWORK
B = 80
NB = EPT // B
N_PAD = 10240
ROWS_PT = N_PAD // 16

W1ROW = 80
W2ROW = 48

_MESH = plsc.VectorSubcoreMesh(core_axis_name="c", subcore_axis_name="s")


def _splat(val, n=16):
    return jnp.full((n,), val, jnp.int32)


_GDN = lax.GatherDimensionNumbers(
    offset_dims=(), collapsed_slice_dims=(0,), start_index_map=(0,))


def _vgather(vec, idx):
    return lax.gather(vec, idx[:, None], _GDN, (1,),
                      mode=lax.GatherScatterMode.PROMISE_IN_BOUNDS)


def _sc1_body(src_hbm, dst_hbm, elp_hbm, erp_hbm, ftp_hbm, zero_hbm,
              part_hbm,
              src_all, dst_all, elbuf, erbuf, ftbuf, msgbuf, wtmp,
              acc_sh, sem):
    c = lax.axis_index("c")
    s = lax.axis_index("s")
    wid = c * 16 + s

    pltpu.sync_copy(src_hbm.at[wid], src_all)
    pltpu.sync_copy(dst_hbm.at[wid], dst_all)

    pltpu.sync_copy(zero_hbm, acc_sh.at[pl.ds(s * ROWS_PT, ROWS_PT)])
    plsc.subcore_barrier()

    def edge(e, _):
        iot = lax.iota(jnp.int32, 16)
        elv = elbuf[e]
        erv = erbuf[e]
        x = elv + erv
        x = jnp.where(x >= 0.0, x, x * NEG_SLOPE)
        w = jnp.exp(x)
        wz = jnp.where(iot < H1, w, 0.0)
        msgbuf[e, pl.ds(HF1, 16)] = wz
        for k in range(4):
            wb = _vgather(w, iot // F1 + 2 * k)
            ftv = ftbuf[e, pl.ds(16 * k, 16)]
            msgbuf[e, pl.ds(16 * k, 16)] = ftv * wb
        return 0

    def batch(b, _):
        d1 = pltpu.async_copy(elp_hbm.at[src_all.at[b]], elbuf, sem)
        d2 = pltpu.async_copy(erp_hbm.at[dst_all.at[b]], erbuf, sem)
        d3 = pltpu.async_copy(ftp_hbm.at[src_all.at[b]], ftbuf, sem)
        d1.wait()
        d2.wait()
        d3.wait()
        lax.fori_loop(0, B, edge, 0)
        pltpu.sync_copy(msgbuf, acc_sh.at[dst_all.at[b]], add=True)
        return 0

    lax.fori_loop(0, NB, batch, 0)
    plsc.subcore_barrier()

    pltpu.sync_copy(acc_sh.at[pl.ds(s * ROWS_PT, ROWS_PT)],
                    part_hbm.at[c].at[pl.ds(s * ROWS_PT, ROWS_PT)])


_sc1 = pl.kernel(
    _sc1_body,
    out_type=jax.ShapeDtypeStruct((2, N_PAD, W1ROW), jnp.float32),
    mesh=_MESH,
    compiler_params=pltpu.CompilerParams(use_tc_tiling_on_sc=False,
                                         needs_layout_passes=False),
    scratch_types=[
        pltpu.VMEM((NB, B), jnp.int32),
        pltpu.VMEM((NB, B), jnp.int32),
        pltpu.VMEM((B, 16), jnp.float32),
        pltpu.VMEM((B, 16), jnp.float32),
        pltpu.VMEM((B, HF1), jnp.float32),
        pltpu.VMEM((B, W1ROW), jnp.float32),
        pltpu.VMEM((16,), jnp.float32),
        pltpu.VMEM_SHARED((N_PAD, W1ROW), jnp.float32),
        pltpu.SemaphoreType.DMA,
    ],
)


def _sc2_body(src_hbm, dst_hbm, elp_hbm, erp_hbm, ftp_hbm, zero_hbm,
              part_hbm,
              src_all, dst_all, elbuf, erbuf, ftbuf, msgbuf, wtmp,
              acc_sh, sem):
    c = lax.axis_index("c")
    s = lax.axis_index("s")
    wid = c * 16 + s

    pltpu.sync_copy(src_hbm.at[wid], src_all)
    pltpu.sync_copy(dst_hbm.at[wid], dst_all)
    pltpu.sync_copy(zero_hbm, acc_sh.at[pl.ds(s * ROWS_PT, ROWS_PT)])
    plsc.subcore_barrier()

    def edge(e, _):
        zer = lax.iota(jnp.int32, 16) * 0
        elv = elbuf[e]
        erv = erbuf[e]
        x = elv + erv
        x = jnp.where(x >= 0.0, x, x * NEG_SLOPE)
        wb = _vgather(jnp.exp(x), zer)
        for k in range(3):
            ftv = ftbuf[e, pl.ds(16 * k, 16)]
            msgbuf[e, pl.ds(16 * k, 16)] = ftv * wb
        return 0

    def batch(b, _):
        d1 = pltpu.async_copy(elp_hbm.at[src_all.at[b]], elbuf, sem)
        d2 = pltpu.async_copy(erp_hbm.at[dst_all.at[b]], erbuf, sem)
        d3 = pltpu.async_copy(ftp_hbm.at[src_all.at[b]], ftbuf, sem)
        d1.wait()
        d2.wait()
        d3.wait()
        lax.fori_loop(0, B, edge, 0)
        pltpu.sync_copy(msgbuf, acc_sh.at[dst_all.at[b]], add=True)
        return 0

    lax.fori_loop(0, NB, batch, 0)
    plsc.subcore_barrier()
    pltpu.sync_copy(acc_sh.at[pl.ds(s * ROWS_PT, ROWS_PT)],
                    part_hbm.at[c].at[pl.ds(s * ROWS_PT, ROWS_PT)])


_sc2 = pl.kernel(
    _sc2_body,
    out_type=jax.ShapeDtypeStruct((2, N_PAD, W2ROW), jnp.float32),
    mesh=_MESH,
    compiler_params=pltpu.CompilerParams(use_tc_tiling_on_sc=False,
                                         needs_layout_passes=False),
    scratch_types=[
        pltpu.VMEM((NB, B), jnp.int32),
        pltpu.VMEM((NB, B), jnp.int32),
        pltpu.VMEM((B, 16), jnp.float32),
        pltpu.VMEM((B, 16), jnp.float32),
        pltpu.VMEM((B, W2ROW), jnp.float32),
        pltpu.VMEM((B, W2ROW), jnp.float32),
        pltpu.VMEM((16,), jnp.float32),
        pltpu.VMEM_SHARED((N_PAD, W2ROW), jnp.float32),
        pltpu.SemaphoreType.DMA,
    ],
)


RB = 1000


def _pre1_body(h_ref, w1_ref, alp_ref, arp_ref, ft_ref, el_ref, er_ref):
    ft = jnp.dot(h_ref[...], w1_ref[...], preferred_element_type=jnp.float32)
    ft_ref[...] = ft
    el_ref[...] = jnp.dot(ft, alp_ref[...], preferred_element_type=jnp.float32)
    er_ref[...] = jnp.dot(ft, arp_ref[...], preferred_element_type=jnp.float32)


def _mid_body(p0_ref, p1_ref, b1_ref, r8_ref, w2p_ref, c40_ref,
              al2p_ref, ar2p_ref, ft2_ref, el2_ref, er2_ref):
    acc = p0_ref[...] + p1_ref[...]
    num = acc[:, 0:HF1]
    den = acc[:, HF1:HF1 + H1]
    den = jnp.where(den > 0.0, den, 1.0)
    rep = jnp.dot(1.0 / den, r8_ref[...], preferred_element_type=jnp.float32)
    x = num * rep + b1_ref[...]
    x = jnp.where(x > 0.0, x, jnp.exp(x) - 1.0)
    ft2 = jnp.dot(x, w2p_ref[...], preferred_element_type=jnp.float32)
    ft2 = ft2 + c40_ref[...]
    ft2_ref[...] = ft2
    el2_ref[...] = jnp.dot(ft2, al2p_ref[...],
                           preferred_element_type=jnp.float32)
    er2_ref[...] = jnp.dot(ft2, ar2p_ref[...],
                           preferred_element_type=jnp.float32)


def _post_body(q0_ref, q1_ref, b2_ref, out_ref):
    acc = q0_ref[...] + q1_ref[...]
    num = acc[:, 0:F2]
    den = acc[:, F2:F2 + 1]
    den = jnp.where(den > 0.0, den, 1.0)
    out_ref[...] = num / den + b2_ref[...]


def _full(shape):
    return pl.BlockSpec(shape, lambda i: (0,) * len(shape))


def _rows(width):
    return pl.BlockSpec((RB, width), lambda i: (i, 0))


def kernel(h, edge_index, W1, al1, ar1, b1, W2, al2, ar2, b2):
    f32 = jnp.float32
    src2 = edge_index[0].reshape(NWORK, NB, B).astype(jnp.int32)
    dst2 = edge_index[1].reshape(NWORK, NB, B).astype(jnp.int32)

    eye8 = jnp.eye(H1, dtype=f32)
    alp1 = (al1[:, :, None] * eye8[:, None, :]).reshape(HF1, H1)
    arp1 = (ar1[:, :, None] * eye8[:, None, :]).reshape(HF1, H1)
    alp1 = jnp.pad(alp1, ((0, 0), (0, 16 - H1)))
    arp1 = jnp.pad(arp1, ((0, 0), (0, 16 - H1)))

    r8 = jnp.repeat(eye8, F1, axis=1)
    b1f = b1.reshape(1, HF1)

    w2p = jnp.pad(W2, ((0, 0), (0, W2ROW - F2)))
    c40 = jnp.zeros((1, W2ROW), f32).at[0, F2].set(1.0)
    al2p = jnp.zeros((W2ROW, 16), f32).at[0:F2, 0].set(al2.reshape(F2))
    ar2p = jnp.zeros((W2ROW, 16), f32).at[0:F2, 0].set(ar2.reshape(F2))
    b2f = b2.reshape(1, F2)

    zh1 = jnp.zeros((ROWS_PT, W1ROW), f32)
    zh2 = jnp.zeros((ROWS_PT, W2ROW), f32)

    grid = (N // RB,)
    ftp, elp, erp = pl.pallas_call(
        _pre1_body,
        grid=grid,
        in_specs=[_rows(D_IN), _full((D_IN, HF1)), _full((HF1, 16)),
                  _full((HF1, 16))],
        out_specs=[_rows(HF1), _rows(16), _rows(16)],
        out_shape=[jax.ShapeDtypeStruct((N, HF1), f32),
                   jax.ShapeDtypeStruct((N, 16), f32),
                   jax.ShapeDtypeStruct((N, 16), f32)],
    )(h, W1, alp1, arp1)

    part1 = _sc1(src2, dst2, elp, erp, ftp, zh1)


    ftp2, elp2, erp2 = pl.pallas_call(
        _mid_body,
        grid=grid,
        in_specs=[_rows(W1ROW), _rows(W1ROW), _full((1, HF1)),
                  _full((H1, HF1)), _full((HF1, W2ROW)), _full((1, W2ROW)),
                  _full((W2ROW, 16)), _full((W2ROW, 16))],
        out_specs=[_rows(W2ROW), _rows(16), _rows(16)],
        out_shape=[jax.ShapeDtypeStruct((N, W2ROW), f32),
                   jax.ShapeDtypeStruct((N, 16), f32),
                   jax.ShapeDtypeStruct((N, 16), f32)],
    )(part1[0], part1[1], b1f, r8, w2p, c40, al2p, ar2p)

    part2 = _sc2(src2, dst2, elp2, erp2, ftp2, zh2)

    out = pl.pallas_call(
        _post_body,
        grid=grid,
        in_specs=[_rows(W2ROW), _rows(W2ROW), _full((1, F2))],
        out_specs=_rows(F2),
        out_shape=jax.ShapeDtypeStruct((N, F2), f32),
    )(part2[0], part2[1], b2f)

    return out


# speedup vs baseline: 64.7362x; 1.3492x over previous
import functools

import jax
import jax.numpy as jnp
from jax import lax
from jax.experimental import pallas as pl
from jax.experimental.pallas import tpu as pltpu
from jax.experimental.pallas import tpu_sc as plsc

N = 10000
E = 320000
D_IN = 128
H1, F1 = 8, 8
HF1 = H1 * F1
F2 = 40
NEG_SLOPE = 0.2

NWORK = 32
EPT = E // NWORK
B = 80
NB = EPT // B
N_PAD = 10240
ROWS_PT = N_PAD // 16

W1ROW = 80
W2ROW = 48

_MESH = plsc.VectorSubcoreMesh(core_axis_name="c", subcore_axis_name="s")


def _splat(val, n=16):
    return jnp.full((n,), val, jnp.int32)


_GDN = lax.GatherDimensionNumbers(
    offset_dims=(), collapsed_slice_dims=(0,), start_index_map=(0,))


def _vgather(vec, idx):
    return lax.gather(vec, idx[:, None], _GDN, (1,),
                      mode=lax.GatherScatterMode.PROMISE_IN_BOUNDS)


def _sc1_body(src_hbm, dst_hbm, elp_hbm, erp_hbm, ftp_hbm, zero_hbm,
              part_hbm,
              src_all, dst_all, elbuf, erbuf, ftbuf, msgbuf,
              acc_sh, gsem0, gsem1, ssem0, ssem1):
    c = lax.axis_index("c")
    s = lax.axis_index("s")
    wid = c * 16 + s
    gsem = (gsem0, gsem1)
    ssem = (ssem0, ssem1)

    pltpu.sync_copy(src_hbm.at[wid], src_all)
    pltpu.sync_copy(dst_hbm.at[wid], dst_all)

    pltpu.sync_copy(zero_hbm, acc_sh.at[pl.ds(s * ROWS_PT, ROWS_PT)])
    plsc.subcore_barrier()

    def g_start(p, b):
        pltpu.async_copy(elp_hbm.at[src_all.at[b]], elbuf.at[p], gsem[p])
        pltpu.async_copy(erp_hbm.at[dst_all.at[b]], erbuf.at[p], gsem[p])
        pltpu.async_copy(ftp_hbm.at[src_all.at[b]], ftbuf.at[p], gsem[p])

    def g_wait(p):
        pltpu.make_async_copy(elp_hbm.at[src_all.at[0]], elbuf.at[p],
                              gsem[p]).wait()
        pltpu.make_async_copy(erp_hbm.at[dst_all.at[0]], erbuf.at[p],
                              gsem[p]).wait()
        pltpu.make_async_copy(ftp_hbm.at[src_all.at[0]], ftbuf.at[p],
                              gsem[p]).wait()

    def s_start(p, b):
        pltpu.async_copy(msgbuf.at[p], acc_sh.at[dst_all.at[b]], ssem[p],
                         add=True)

    def s_wait(p, b):
        pltpu.make_async_copy(msgbuf.at[p], acc_sh.at[dst_all.at[b]],
                              ssem[p]).wait()

    def make_edge(p):
        def edge(e, _):
            iot = lax.iota(jnp.int32, 16)
            x = elbuf[p, e] + erbuf[p, e]
            x = jnp.where(x >= 0.0, x, x * NEG_SLOPE)
            w = jnp.exp(x)
            wz = jnp.where(iot < H1, w, 0.0)
            msgbuf[p, e, pl.ds(HF1, 16)] = wz
            for k in range(4):
                wb = _vgather(w, iot // F1 + 2 * k)
                ftv = ftbuf[p, e, pl.ds(16 * k, 16)]
                msgbuf[p, e, pl.ds(16 * k, 16)] = ftv * wb
            return 0
        return edge

    g_start(0, 0)

    def pair(i, _):
        b0 = 2 * i
        b1 = b0 + 1
        g_wait(0)
        g_start(1, b1)
        lax.fori_loop(0, B, make_edge(0), 0)
        s_start(0, b0)
        g_wait(1)
        g_start(0, b0 + 2)
        lax.fori_loop(0, B, make_edge(1), 0)
        s_start(1, b1)
        s_wait(0, b0)
        s_wait(1, b1)
        return 0

    lax.fori_loop(0, NB // 2, pair, 0)
    g_wait(0)
    lax.fori_loop(0, B, make_edge(0), 0)
    pltpu.sync_copy(msgbuf.at[0], acc_sh.at[dst_all.at[NB - 1]], add=True)
    plsc.subcore_barrier()

    pltpu.sync_copy(acc_sh.at[pl.ds(s * ROWS_PT, ROWS_PT)],
                    part_hbm.at[c].at[pl.ds(s * ROWS_PT, ROWS_PT)])


_sc1 = pl.kernel(
    _sc1_body,
    out_type=jax.ShapeDtypeStruct((2, N_PAD, W1ROW), jnp.float32),
    mesh=_MESH,
    compiler_params=pltpu.CompilerParams(use_tc_tiling_on_sc=False,
                                         needs_layout_passes=False),
    scratch_types=[
        pltpu.VMEM((NB, B), jnp.int32),
        pltpu.VMEM((NB, B), jnp.int32),
        pltpu.VMEM((2, B, 16), jnp.float32),
        pltpu.VMEM((2, B, 16), jnp.float32),
        pltpu.VMEM((2, B, HF1), jnp.float32),
        pltpu.VMEM((2, B, W1ROW), jnp.float32),
        pltpu.VMEM_SHARED((N_PAD, W1ROW), jnp.float32),
        pltpu.SemaphoreType.DMA,
        pltpu.SemaphoreType.DMA,
        pltpu.SemaphoreType.DMA,
        pltpu.SemaphoreType.DMA,
    ],
)


def _sc2_body(src_hbm, dst_hbm, elp_hbm, erp_hbm, ftp_hbm, zero_hbm,
              part_hbm,
              src_all, dst_all, elbuf, erbuf, ftbuf, msgbuf,
              acc_sh, gsem0, gsem1, ssem0, ssem1):
    c = lax.axis_index("c")
    s = lax.axis_index("s")
    wid = c * 16 + s
    gsem = (gsem0, gsem1)
    ssem = (ssem0, ssem1)

    pltpu.sync_copy(src_hbm.at[wid], src_all)
    pltpu.sync_copy(dst_hbm.at[wid], dst_all)
    pltpu.sync_copy(zero_hbm, acc_sh.at[pl.ds(s * ROWS_PT, ROWS_PT)])
    plsc.subcore_barrier()

    def g_start(p, b):
        pltpu.async_copy(elp_hbm.at[src_all.at[b]], elbuf.at[p], gsem[p])
        pltpu.async_copy(erp_hbm.at[dst_all.at[b]], erbuf.at[p], gsem[p])
        pltpu.async_copy(ftp_hbm.at[src_all.at[b]], ftbuf.at[p], gsem[p])

    def g_wait(p):
        pltpu.make_async_copy(elp_hbm.at[src_all.at[0]], elbuf.at[p],
                              gsem[p]).wait()
        pltpu.make_async_copy(erp_hbm.at[dst_all.at[0]], erbuf.at[p],
                              gsem[p]).wait()
        pltpu.make_async_copy(ftp_hbm.at[src_all.at[0]], ftbuf.at[p],
                              gsem[p]).wait()

    def s_start(p, b):
        pltpu.async_copy(msgbuf.at[p], acc_sh.at[dst_all.at[b]], ssem[p],
                         add=True)

    def s_wait(p, b):
        pltpu.make_async_copy(msgbuf.at[p], acc_sh.at[dst_all.at[b]],
                              ssem[p]).wait()

    def make_edge(p):
        def edge(e, _):
            zer = lax.iota(jnp.int32, 16) * 0
            x = elbuf[p, e] + erbuf[p, e]
            x = jnp.where(x >= 0.0, x, x * NEG_SLOPE)
            wb = _vgather(jnp.exp(x), zer)
            for k in range(3):
                ftv = ftbuf[p, e, pl.ds(16 * k, 16)]
                msgbuf[p, e, pl.ds(16 * k, 16)] = ftv * wb
            return 0
        return edge

    g_start(0, 0)

    def pair(i, _):
        b0 = 2 * i
        b1 = b0 + 1
        g_wait(0)
        g_start(1, b1)
        lax.fori_loop(0, B, make_edge(0), 0)
        s_start(0, b0)
        g_wait(1)
        g_start(0, b0 + 2)
        lax.fori_loop(0, B, make_edge(1), 0)
        s_start(1, b1)
        s_wait(0, b0)
        s_wait(1, b1)
        return 0

    lax.fori_loop(0, NB // 2, pair, 0)
    g_wait(0)
    lax.fori_loop(0, B, make_edge(0), 0)
    pltpu.sync_copy(msgbuf.at[0], acc_sh.at[dst_all.at[NB - 1]], add=True)
    plsc.subcore_barrier()
    pltpu.sync_copy(acc_sh.at[pl.ds(s * ROWS_PT, ROWS_PT)],
                    part_hbm.at[c].at[pl.ds(s * ROWS_PT, ROWS_PT)])


_sc2 = pl.kernel(
    _sc2_body,
    out_type=jax.ShapeDtypeStruct((2, N_PAD, W2ROW), jnp.float32),
    mesh=_MESH,
    compiler_params=pltpu.CompilerParams(use_tc_tiling_on_sc=False,
                                         needs_layout_passes=False),
    scratch_types=[
        pltpu.VMEM((NB, B), jnp.int32),
        pltpu.VMEM((NB, B), jnp.int32),
        pltpu.VMEM((2, B, 16), jnp.float32),
        pltpu.VMEM((2, B, 16), jnp.float32),
        pltpu.VMEM((2, B, W2ROW), jnp.float32),
        pltpu.VMEM((2, B, W2ROW), jnp.float32),
        pltpu.VMEM_SHARED((N_PAD, W2ROW), jnp.float32),
        pltpu.SemaphoreType.DMA,
        pltpu.SemaphoreType.DMA,
        pltpu.SemaphoreType.DMA,
        pltpu.SemaphoreType.DMA,
    ],
)


RB = 1000


def _pre1_body(h_ref, w1_ref, alp_ref, arp_ref, ft_ref, el_ref, er_ref):
    ft = jnp.dot(h_ref[...], w1_ref[...], preferred_element_type=jnp.float32)
    ft_ref[...] = ft
    el_ref[...] = jnp.dot(ft, alp_ref[...], preferred_element_type=jnp.float32)
    er_ref[...] = jnp.dot(ft, arp_ref[...], preferred_element_type=jnp.float32)


def _mid_body(p0_ref, p1_ref, b1_ref, r8_ref, w2p_ref, c40_ref,
              al2p_ref, ar2p_ref, ft2_ref, el2_ref, er2_ref):
    acc = p0_ref[...] + p1_ref[...]
    num = acc[:, 0:HF1]
    den = acc[:, HF1:HF1 + H1]
    den = jnp.where(den > 0.0, den, 1.0)
    rep = jnp.dot(1.0 / den, r8_ref[...], preferred_element_type=jnp.float32)
    x = num * rep + b1_ref[...]
    x = jnp.where(x > 0.0, x, jnp.exp(x) - 1.0)
    ft2 = jnp.dot(x, w2p_ref[...], preferred_element_type=jnp.float32)
    ft2 = ft2 + c40_ref[...]
    ft2_ref[...] = ft2
    el2_ref[...] = jnp.dot(ft2, al2p_ref[...],
                           preferred_element_type=jnp.float32)
    er2_ref[...] = jnp.dot(ft2, ar2p_ref[...],
                           preferred_element_type=jnp.float32)


def _post_body(q0_ref, q1_ref, b2_ref, out_ref):
    acc = q0_ref[...] + q1_ref[...]
    num = acc[:, 0:F2]
    den = acc[:, F2:F2 + 1]
    den = jnp.where(den > 0.0, den, 1.0)
    out_ref[...] = num / den + b2_ref[...]


def _full(shape):
    return pl.BlockSpec(shape, lambda i: (0,) * len(shape))


def _rows(width):
    return pl.BlockSpec((RB, width), lambda i: (i, 0))


def kernel(h, edge_index, W1, al1, ar1, b1, W2, al2, ar2, b2):
    f32 = jnp.float32
    src2 = edge_index[0].reshape(NWORK, NB, B).astype(jnp.int32)
    dst2 = edge_index[1].reshape(NWORK, NB, B).astype(jnp.int32)

    eye8 = jnp.eye(H1, dtype=f32)
    alp1 = (al1[:, :, None] * eye8[:, None, :]).reshape(HF1, H1)
    arp1 = (ar1[:, :, None] * eye8[:, None, :]).reshape(HF1, H1)
    alp1 = jnp.pad(alp1, ((0, 0), (0, 16 - H1)))
    arp1 = jnp.pad(arp1, ((0, 0), (0, 16 - H1)))

    r8 = jnp.repeat(eye8, F1, axis=1)
    b1f = b1.reshape(1, HF1)

    w2p = jnp.pad(W2, ((0, 0), (0, W2ROW - F2)))
    c40 = jnp.zeros((1, W2ROW), f32).at[0, F2].set(1.0)
    al2p = jnp.zeros((W2ROW, 16), f32).at[0:F2, 0].set(al2.reshape(F2))
    ar2p = jnp.zeros((W2ROW, 16), f32).at[0:F2, 0].set(ar2.reshape(F2))
    b2f = b2.reshape(1, F2)

    zh1 = jnp.zeros((ROWS_PT, W1ROW), f32)
    zh2 = jnp.zeros((ROWS_PT, W2ROW), f32)

    grid = (N // RB,)
    ftp, elp, erp = pl.pallas_call(
        _pre1_body,
        grid=grid,
        in_specs=[_rows(D_IN), _full((D_IN, HF1)), _full((HF1, 16)),
                  _full((HF1, 16))],
        out_specs=[_rows(HF1), _rows(16), _rows(16)],
        out_shape=[jax.ShapeDtypeStruct((N, HF1), f32),
                   jax.ShapeDtypeStruct((N, 16), f32),
                   jax.ShapeDtypeStruct((N, 16), f32)],
    )(h, W1, alp1, arp1)

    part1 = _sc1(src2, dst2, elp, erp, ftp, zh1)


    ftp2, elp2, erp2 = pl.pallas_call(
        _mid_body,
        grid=grid,
        in_specs=[_rows(W1ROW), _rows(W1ROW), _full((1, HF1)),
                  _full((H1, HF1)), _full((HF1, W2ROW)), _full((1, W2ROW)),
                  _full((W2ROW, 16)), _full((W2ROW, 16))],
        out_specs=[_rows(W2ROW), _rows(16), _rows(16)],
        out_shape=[jax.ShapeDtypeStruct((N, W2ROW), f32),
                   jax.ShapeDtypeStruct((N, 16), f32),
                   jax.ShapeDtypeStruct((N, 16), f32)],
    )(part1[0], part1[1], b1f, r8, w2p, c40, al2p, ar2p)

    part2 = _sc2(src2, dst2, elp2, erp2, ftp2, zh2)

    out = pl.pallas_call(
        _post_body,
        grid=grid,
        in_specs=[_rows(W2ROW), _rows(W2ROW), _full((1, F2))],
        out_specs=_rows(F2),
        out_shape=jax.ShapeDtypeStruct((N, F2), f32),
    )(part2[0], part2[1], b2f)

    return out


# speedup vs baseline: 120.2802x; 1.8580x over previous
import functools

import jax
import jax.numpy as jnp
from jax import lax
from jax.experimental import pallas as pl
from jax.experimental.pallas import tpu as pltpu
from jax.experimental.pallas import tpu_sc as plsc

N = 10000
E = 320000
D_IN = 128
H1, F1 = 8, 8
HF1 = H1 * F1
F2 = 40
NEG_SLOPE = 0.2

NWORK = 32
EPT = E // NWORK
B = 80
NB = EPT // B
N_PAD = 10240
ROWS_PT = N_PAD // 16

W1ROW = 80
W2ROW = 48

_MESH = plsc.VectorSubcoreMesh(core_axis_name="c", subcore_axis_name="s")


def _splat(val, n=16):
    return jnp.full((n,), val, jnp.int32)


_GDN = lax.GatherDimensionNumbers(
    offset_dims=(), collapsed_slice_dims=(0,), start_index_map=(0,))


def _vgather(vec, idx):
    return lax.gather(vec, idx[:, None], _GDN, (1,),
                      mode=lax.GatherScatterMode.PROMISE_IN_BOUNDS)


def _sc1_body(src_hbm, dst_hbm, elp_hbm, erp_hbm, ftp_hbm, zero_hbm,
              part_hbm,
              src_all, dst_all, elbuf, erbuf, ftbuf, msgbuf,
              acc_sh, gsem0, gsem1, ssem0, ssem1):
    c = lax.axis_index("c")
    s = lax.axis_index("s")
    wid = c * 16 + s
    gsem = (gsem0, gsem1)
    ssem = (ssem0, ssem1)

    pltpu.sync_copy(src_hbm.at[wid], src_all)
    pltpu.sync_copy(dst_hbm.at[wid], dst_all)

    pltpu.sync_copy(zero_hbm, acc_sh.at[pl.ds(s * ROWS_PT, ROWS_PT)])
    plsc.subcore_barrier()

    def g_start(p, b):
        pltpu.async_copy(elp_hbm.at[src_all.at[b]], elbuf.at[p], gsem[p])
        pltpu.async_copy(erp_hbm.at[dst_all.at[b]], erbuf.at[p], gsem[p])
        pltpu.async_copy(ftp_hbm.at[src_all.at[b]], ftbuf.at[p], gsem[p])

    def g_wait(p):
        pltpu.make_async_copy(elp_hbm.at[src_all.at[0]], elbuf.at[p],
                              gsem[p]).wait()
        pltpu.make_async_copy(erp_hbm.at[dst_all.at[0]], erbuf.at[p],
                              gsem[p]).wait()
        pltpu.make_async_copy(ftp_hbm.at[src_all.at[0]], ftbuf.at[p],
                              gsem[p]).wait()

    def s_start(p, b):
        pltpu.async_copy(msgbuf.at[p], acc_sh.at[dst_all.at[b]], ssem[p],
                         add=True)

    def s_wait(p, b):
        pltpu.make_async_copy(msgbuf.at[p], acc_sh.at[dst_all.at[b]],
                              ssem[p]).wait()

    def run_edges(p):
        def edge(e):
            iot = lax.iota(jnp.int32, 16)
            x = elbuf[p, e] + erbuf[p, e]
            x = jnp.where(x >= 0.0, x, x * NEG_SLOPE)
            w = jnp.exp(x)
            wz = jnp.where(iot < H1, w, 0.0)
            msgbuf[p, e, pl.ds(HF1, 16)] = wz
            for k in range(4):
                wb = _vgather(w, iot // F1 + 2 * k)
                ftv = ftbuf[p, e, pl.ds(16 * k, 16)]
                msgbuf[p, e, pl.ds(16 * k, 16)] = ftv * wb
        plsc.parallel_loop(0, B, unroll=4)(edge)

    g_start(0, 0)

    def pair(i, _):
        b0 = 2 * i
        b1 = b0 + 1
        g_wait(0)
        g_start(1, b1)
        run_edges(0)
        s_start(0, b0)
        g_wait(1)
        g_start(0, b0 + 2)
        run_edges(1)
        s_start(1, b1)
        s_wait(0, b0)
        s_wait(1, b1)
        return 0

    lax.fori_loop(0, NB // 2, pair, 0)
    g_wait(0)
    run_edges(0)
    pltpu.sync_copy(msgbuf.at[0], acc_sh.at[dst_all.at[NB - 1]], add=True)
    plsc.subcore_barrier()

    pltpu.sync_copy(acc_sh.at[pl.ds(s * ROWS_PT, ROWS_PT)],
                    part_hbm.at[c].at[pl.ds(s * ROWS_PT, ROWS_PT)])


_sc1 = pl.kernel(
    _sc1_body,
    out_type=jax.ShapeDtypeStruct((2, N_PAD, W1ROW), jnp.float32),
    mesh=_MESH,
    compiler_params=pltpu.CompilerParams(use_tc_tiling_on_sc=False,
                                         needs_layout_passes=False),
    scratch_types=[
        pltpu.VMEM((NB, B), jnp.int32),
        pltpu.VMEM((NB, B), jnp.int32),
        pltpu.VMEM((2, B, 16), jnp.float32),
        pltpu.VMEM((2, B, 16), jnp.float32),
        pltpu.VMEM((2, B, HF1), jnp.float32),
        pltpu.VMEM((2, B, W1ROW), jnp.float32),
        pltpu.VMEM_SHARED((N_PAD, W1ROW), jnp.float32),
        pltpu.SemaphoreType.DMA,
        pltpu.SemaphoreType.DMA,
        pltpu.SemaphoreType.DMA,
        pltpu.SemaphoreType.DMA,
    ],
)


def _sc2_body(src_hbm, dst_hbm, elp_hbm, erp_hbm, ftp_hbm, zero_hbm,
              part_hbm,
              src_all, dst_all, elbuf, erbuf, ftbuf, msgbuf,
              acc_sh, gsem0, gsem1, ssem0, ssem1):
    c = lax.axis_index("c")
    s = lax.axis_index("s")
    wid = c * 16 + s
    gsem = (gsem0, gsem1)
    ssem = (ssem0, ssem1)

    pltpu.sync_copy(src_hbm.at[wid], src_all)
    pltpu.sync_copy(dst_hbm.at[wid], dst_all)
    pltpu.sync_copy(zero_hbm, acc_sh.at[pl.ds(s * ROWS_PT, ROWS_PT)])
    plsc.subcore_barrier()

    def g_start(p, b):
        pltpu.async_copy(elp_hbm.at[src_all.at[b]], elbuf.at[p], gsem[p])
        pltpu.async_copy(erp_hbm.at[dst_all.at[b]], erbuf.at[p], gsem[p])
        pltpu.async_copy(ftp_hbm.at[src_all.at[b]], ftbuf.at[p], gsem[p])

    def g_wait(p):
        pltpu.make_async_copy(elp_hbm.at[src_all.at[0]], elbuf.at[p],
                              gsem[p]).wait()
        pltpu.make_async_copy(erp_hbm.at[dst_all.at[0]], erbuf.at[p],
                              gsem[p]).wait()
        pltpu.make_async_copy(ftp_hbm.at[src_all.at[0]], ftbuf.at[p],
                              gsem[p]).wait()

    def s_start(p, b):
        pltpu.async_copy(msgbuf.at[p], acc_sh.at[dst_all.at[b]], ssem[p],
                         add=True)

    def s_wait(p, b):
        pltpu.make_async_copy(msgbuf.at[p], acc_sh.at[dst_all.at[b]],
                              ssem[p]).wait()

    def run_edges(p):
        def edge(e):
            zer = lax.iota(jnp.int32, 16) * 0
            x = elbuf[p, e] + erbuf[p, e]
            x = jnp.where(x >= 0.0, x, x * NEG_SLOPE)
            wb = _vgather(jnp.exp(x), zer)
            for k in range(3):
                ftv = ftbuf[p, e, pl.ds(16 * k, 16)]
                msgbuf[p, e, pl.ds(16 * k, 16)] = ftv * wb
        plsc.parallel_loop(0, B, unroll=4)(edge)

    g_start(0, 0)

    def pair(i, _):
        b0 = 2 * i
        b1 = b0 + 1
        g_wait(0)
        g_start(1, b1)
        run_edges(0)
        s_start(0, b0)
        g_wait(1)
        g_start(0, b0 + 2)
        run_edges(1)
        s_start(1, b1)
        s_wait(0, b0)
        s_wait(1, b1)
        return 0

    lax.fori_loop(0, NB // 2, pair, 0)
    g_wait(0)
    run_edges(0)
    pltpu.sync_copy(msgbuf.at[0], acc_sh.at[dst_all.at[NB - 1]], add=True)
    plsc.subcore_barrier()
    pltpu.sync_copy(acc_sh.at[pl.ds(s * ROWS_PT, ROWS_PT)],
                    part_hbm.at[c].at[pl.ds(s * ROWS_PT, ROWS_PT)])


_sc2 = pl.kernel(
    _sc2_body,
    out_type=jax.ShapeDtypeStruct((2, N_PAD, W2ROW), jnp.float32),
    mesh=_MESH,
    compiler_params=pltpu.CompilerParams(use_tc_tiling_on_sc=False,
                                         needs_layout_passes=False),
    scratch_types=[
        pltpu.VMEM((NB, B), jnp.int32),
        pltpu.VMEM((NB, B), jnp.int32),
        pltpu.VMEM((2, B, 16), jnp.float32),
        pltpu.VMEM((2, B, 16), jnp.float32),
        pltpu.VMEM((2, B, W2ROW), jnp.float32),
        pltpu.VMEM((2, B, W2ROW), jnp.float32),
        pltpu.VMEM_SHARED((N_PAD, W2ROW), jnp.float32),
        pltpu.SemaphoreType.DMA,
        pltpu.SemaphoreType.DMA,
        pltpu.SemaphoreType.DMA,
        pltpu.SemaphoreType.DMA,
    ],
)


RB = 1000


def _pre1_body(h_ref, w1_ref, alp_ref, arp_ref, ft_ref, el_ref, er_ref):
    ft = jnp.dot(h_ref[...], w1_ref[...], preferred_element_type=jnp.float32)
    ft_ref[...] = ft
    el_ref[...] = jnp.dot(ft, alp_ref[...], preferred_element_type=jnp.float32)
    er_ref[...] = jnp.dot(ft, arp_ref[...], preferred_element_type=jnp.float32)


def _mid_body(p0_ref, p1_ref, b1_ref, r8_ref, w2p_ref, c40_ref,
              al2p_ref, ar2p_ref, ft2_ref, el2_ref, er2_ref):
    acc = p0_ref[...] + p1_ref[...]
    num = acc[:, 0:HF1]
    den = acc[:, HF1:HF1 + H1]
    den = jnp.where(den > 0.0, den, 1.0)
    rep = jnp.dot(1.0 / den, r8_ref[...], preferred_element_type=jnp.float32)
    x = num * rep + b1_ref[...]
    x = jnp.where(x > 0.0, x, jnp.exp(x) - 1.0)
    ft2 = jnp.dot(x, w2p_ref[...], preferred_element_type=jnp.float32)
    ft2 = ft2 + c40_ref[...]
    ft2_ref[...] = ft2
    el2_ref[...] = jnp.dot(ft2, al2p_ref[...],
                           preferred_element_type=jnp.float32)
    er2_ref[...] = jnp.dot(ft2, ar2p_ref[...],
                           preferred_element_type=jnp.float32)


def _post_body(q0_ref, q1_ref, b2_ref, out_ref):
    acc = q0_ref[...] + q1_ref[...]
    num = acc[:, 0:F2]
    den = acc[:, F2:F2 + 1]
    den = jnp.where(den > 0.0, den, 1.0)
    out_ref[...] = num / den + b2_ref[...]


def _full(shape):
    return pl.BlockSpec(shape, lambda i: (0,) * len(shape))


def _rows(width):
    return pl.BlockSpec((RB, width), lambda i: (i, 0))


def kernel(h, edge_index, W1, al1, ar1, b1, W2, al2, ar2, b2):
    f32 = jnp.float32
    src2 = edge_index[0].reshape(NWORK, NB, B).astype(jnp.int32)
    dst2 = edge_index[1].reshape(NWORK, NB, B).astype(jnp.int32)

    eye8 = jnp.eye(H1, dtype=f32)
    alp1 = (al1[:, :, None] * eye8[:, None, :]).reshape(HF1, H1)
    arp1 = (ar1[:, :, None] * eye8[:, None, :]).reshape(HF1, H1)
    alp1 = jnp.pad(alp1, ((0, 0), (0, 16 - H1)))
    arp1 = jnp.pad(arp1, ((0, 0), (0, 16 - H1)))

    r8 = jnp.repeat(eye8, F1, axis=1)
    b1f = b1.reshape(1, HF1)

    w2p = jnp.pad(W2, ((0, 0), (0, W2ROW - F2)))
    c40 = jnp.zeros((1, W2ROW), f32).at[0, F2].set(1.0)
    al2p = jnp.zeros((W2ROW, 16), f32).at[0:F2, 0].set(al2.reshape(F2))
    ar2p = jnp.zeros((W2ROW, 16), f32).at[0:F2, 0].set(ar2.reshape(F2))
    b2f = b2.reshape(1, F2)

    zh1 = jnp.zeros((ROWS_PT, W1ROW), f32)
    zh2 = jnp.zeros((ROWS_PT, W2ROW), f32)

    grid = (N // RB,)
    ftp, elp, erp = pl.pallas_call(
        _pre1_body,
        grid=grid,
        in_specs=[_rows(D_IN), _full((D_IN, HF1)), _full((HF1, 16)),
                  _full((HF1, 16))],
        out_specs=[_rows(HF1), _rows(16), _rows(16)],
        out_shape=[jax.ShapeDtypeStruct((N, HF1), f32),
                   jax.ShapeDtypeStruct((N, 16), f32),
                   jax.ShapeDtypeStruct((N, 16), f32)],
    )(h, W1, alp1, arp1)

    part1 = _sc1(src2, dst2, elp, erp, ftp, zh1)


    ftp2, elp2, erp2 = pl.pallas_call(
        _mid_body,
        grid=grid,
        in_specs=[_rows(W1ROW), _rows(W1ROW), _full((1, HF1)),
                  _full((H1, HF1)), _full((HF1, W2ROW)), _full((1, W2ROW)),
                  _full((W2ROW, 16)), _full((W2ROW, 16))],
        out_specs=[_rows(W2ROW), _rows(16), _rows(16)],
        out_shape=[jax.ShapeDtypeStruct((N, W2ROW), f32),
                   jax.ShapeDtypeStruct((N, 16), f32),
                   jax.ShapeDtypeStruct((N, 16), f32)],
    )(part1[0], part1[1], b1f, r8, w2p, c40, al2p, ar2p)

    part2 = _sc2(src2, dst2, elp2, erp2, ftp2, zh2)

    out = pl.pallas_call(
        _post_body,
        grid=grid,
        in_specs=[_rows(W2ROW), _rows(W2ROW), _full((1, F2))],
        out_specs=_rows(F2),
        out_shape=jax.ShapeDtypeStruct((N, F2), f32),
    )(part2[0], part2[1], b2f)

    return out
